# baseline (speedup 1.0000x reference)
import jax
import jax.numpy as jnp
from jax.experimental import pallas as pl
from jax.experimental.pallas import tpu as pltpu
from jax.experimental.pallas import tpu_sc as plsc

_N = 10000
_E = 320000
_D = 128

_NC = 2
_NS = 16
_NW = _NC * _NS
_EPT = _E // _NW
_VCH = _EPT // 16
_DEGN = 10240
_CHW = 125
_EPS = _E // _NS
_NCH = _EPS // _CHW
_NBUF = 4
_DH = _D // _NC
_NACC = 10240
_STRIPE = _NACC // _NS
_ZR = 64

_MIN_NORM = 1e-15
_MAXN = 1.0 - 4e-3


def _sc_mesh():
    return plsc.VectorSubcoreMesh(
        core_axis_name="c", subcore_axis_name="s", num_cores=_NC, num_subcores=_NS
    )


def _deg_body(row_hbm, col_hbm, degp_hbm, rowp_hbm, rowbuf, colbuf, degbuf):
    cid = jax.lax.axis_index("c")
    sid = jax.lax.axis_index("s")
    wid = sid * _NC + cid
    base = wid * _EPT
    pltpu.sync_copy(row_hbm.at[pl.ds(base, _EPT)], rowbuf)
    pltpu.sync_copy(col_hbm.at[pl.ds(base, _EPT)], colbuf)

    def zero(i, carry):
        degbuf[pl.ds(i * 16, 16)] = jnp.zeros((16,), jnp.float32)
        return carry

    jax.lax.fori_loop(0, _DEGN // 16, zero, 0)

    ones = jnp.ones((16,), jnp.float32)
    nvec = jnp.full((16,), _N, jnp.int32)

    def step(i, carry):
        r = rowbuf[pl.ds(i * 16, 16)]
        c = colbuf[pl.ds(i * 16, 16)]
        m = r != c
        plsc.addupdate_scatter(degbuf, [r], ones, mask=m)
        rowbuf[pl.ds(i * 16, 16)] = jnp.where(m, r, nvec)
        return carry

    jax.lax.fori_loop(0, _VCH, step, 0)

    pltpu.sync_copy(degbuf, degp_hbm.at[wid])
    pltpu.sync_copy(rowbuf, rowp_hbm.at[pl.ds(base, _EPT)])


def _deg_call(row, col):
    f = pl.kernel(
        _deg_body,
        out_type=[
            jax.ShapeDtypeStruct((_NW, _DEGN), jnp.float32),
            jax.ShapeDtypeStruct((_E,), jnp.int32),
        ],
        mesh=_sc_mesh(),
        scratch_types=[
            pltpu.VMEM((_EPT,), jnp.int32),
            pltpu.VMEM((_EPT,), jnp.int32),
            pltpu.VMEM((_DEGN,), jnp.float32),
        ],
        compiler_params=pltpu.CompilerParams(needs_layout_passes=False),
    )
    return f(row, col)


def _agg_body(y_hbm, rowp_hbm, col_hbm, out_hbm,
              rowbuf, colbuf, bufs, zbuf, acc, gsem, ssem):
    cid = jax.lax.axis_index("c")
    sid = jax.lax.axis_index("s")
    pltpu.sync_copy(rowp_hbm.at[sid], rowbuf)
    pltpu.sync_copy(col_hbm.at[sid], colbuf)

    def zrow(i, carry):
        def zcol(j, carry2):
            zbuf[i, pl.ds(j * 16, 16)] = jnp.zeros((16,), jnp.float32)
            return carry2
        return jax.lax.fori_loop(0, _DH // 16, zcol, carry)

    jax.lax.fori_loop(0, _ZR, zrow, 0)
    for k in range(_STRIPE // _ZR):
        pltpu.sync_copy(zbuf, acc.at[pl.ds(sid * _STRIPE + k * _ZR, _ZR)])
    plsc.subcore_barrier()

    def gather(g, j):
        return pltpu.make_async_copy(
            y_hbm.at[cid].at[colbuf.at[g]], bufs[j], gsem.at[j]
        )

    def scat_start(g, j):
        pltpu.async_copy(bufs[j], acc.at[rowbuf.at[g]], ssem.at[j], add=True)

    def scat_wait(g, j):
        pltpu.make_async_copy(bufs[j], acc.at[rowbuf.at[g]], ssem.at[j]).wait()

    for j in range(_NBUF):
        gather(j, j).start()

    def body(h, carry):
        g0 = _NBUF * h
        for j in range(_NBUF):
            g = g0 + j
            jn = (j + 1) % _NBUF
            gather(g, j).wait()
            scat_start(g, j)

            @pl.when((g >= _NBUF - 1) & (g + 1 < _NCH))
            def _():
                scat_wait(g - (_NBUF - 1), jn)
                gather(g + 1, jn).start()

        return carry

    jax.lax.fori_loop(0, _NCH // _NBUF, body, 0)
    for j in range(_NBUF):
        scat_wait(0, j)

    plsc.subcore_barrier()
    pltpu.sync_copy(
        acc.at[pl.ds(sid * _STRIPE, _STRIPE)],
        out_hbm.at[cid, pl.ds(sid * _STRIPE, _STRIPE)],
    )


def _agg_call(ysplit, rowp, col):
    f = pl.kernel(
        _agg_body,
        out_type=[jax.ShapeDtypeStruct((_NC, _NACC, _DH), jnp.float32)],
        mesh=_sc_mesh(),
        scratch_types=[
            pltpu.VMEM((_NCH, _CHW), jnp.int32),
            pltpu.VMEM((_NCH, _CHW), jnp.int32),
            [pltpu.VMEM((_CHW, _DH), jnp.float32) for _ in range(_NBUF)],
            pltpu.VMEM((_ZR, _DH), jnp.float32),
            pltpu.VMEM_SHARED((_NACC, _DH), jnp.float32),
            pltpu.SemaphoreType.DMA((_NBUF,)),
            pltpu.SemaphoreType.DMA((_NBUF,)),
        ],
        compiler_params=pltpu.CompilerParams(
            needs_layout_passes=False, use_tc_tiling_on_sc=False
        ),
    )
    return f(ysplit, rowp, col)


def _nrm(v):
    return jnp.sqrt(jnp.clip(jnp.sum(v * v, axis=-1, keepdims=True), 1e-30, None))


def _artanh(v):
    v = jnp.clip(v, -1.0 + 1e-7, 1.0 - 1e-7)
    return 0.5 * (jnp.log1p(v) - jnp.log1p(-v))


def _proj(v):
    n = jnp.clip(_nrm(v), _MIN_NORM, None)
    return jnp.where(n > _MAXN, v / n * _MAXN, v)


def _expmap0(u):
    un = jnp.clip(_nrm(u), _MIN_NORM, None)
    return jnp.tanh(un) * u / un


def _logmap0(p):
    pn = jnp.clip(_nrm(p), _MIN_NORM, None)
    return _artanh(pn) * p / pn


def _mobius_add(a, bb):
    a2 = jnp.sum(a * a, axis=-1, keepdims=True)
    b2 = jnp.sum(bb * bb, axis=-1, keepdims=True)
    ab = jnp.sum(a * bb, axis=-1, keepdims=True)
    num = (1.0 + 2.0 * ab + b2) * a + (1.0 - a2) * bb
    den = 1.0 + 2.0 * ab + a2 * b2
    return num / jnp.clip(den, _MIN_NORM, None)


_RB = 1000


def _tc_a_body(x_ref, w_ref, b_ref, degpt_ref, y_ref):
    x = x_ref[...]
    w = w_ref[...]
    b = b_ref[...]
    mx = jax.lax.dot_general(
        x, w, (((1,), (1,)), ((), ())), preferred_element_type=jnp.float32
    )
    xn = jnp.clip(_nrm(x), _MIN_NORM, None)
    mxn = jnp.clip(_nrm(mx), _MIN_NORM, None)
    res = jnp.tanh(mxn / xn * _artanh(xn)) * mx / mxn
    iszero = jnp.sum(jnp.abs(mx), axis=-1, keepdims=True) == 0.0
    res = jnp.where(iszero, jnp.zeros_like(res), res)
    res = _proj(res)
    hb = _proj(_expmap0(b))
    h = _proj(_mobius_add(res, hb))
    xt = _logmap0(h)
    deg = jnp.sum(degpt_ref[...], axis=1, keepdims=True) + 1.0
    dis = jax.lax.rsqrt(deg)
    y = dis * xt
    y_ref[...] = jnp.stack([y[:, :_DH], y[:, _DH:]], axis=0)


def _tc_a(x, w, b2, degpt):
    return pl.pallas_call(
        _tc_a_body,
        grid=(_N // _RB,),
        in_specs=[
            pl.BlockSpec((_RB, _D), lambda i: (i, 0)),
            pl.BlockSpec((_D, _D), lambda i: (0, 0)),
            pl.BlockSpec((1, _D), lambda i: (0, 0)),
            pl.BlockSpec((_RB, _NW), lambda i: (i, 0)),
        ],
        out_specs=pl.BlockSpec((_NC, _RB, _DH), lambda i: (0, i, 0)),
        out_shape=jax.ShapeDtypeStruct((_NC, _N, _DH), jnp.float32),
    )(x, w, b2, degpt)


def _tc_c_body(a0_ref, a1_ref, y0_ref, y1_ref, degpt_ref, o_ref):
    s = (jnp.concatenate([a0_ref[0], a1_ref[0]], axis=-1)
         + jnp.concatenate([y0_ref[0], y1_ref[0]], axis=-1))
    deg = jnp.sum(degpt_ref[...], axis=1, keepdims=True) + 1.0
    dis = jax.lax.rsqrt(deg)
    st = dis * s
    o1 = _proj(_expmap0(st))
    a = _logmap0(o1)
    a = jnp.where(a >= 0.0, a, 0.01 * a)
    o_ref[...] = _proj(_expmap0(a))


def _tc_c(aggp, ysplit, degpt):
    return pl.pallas_call(
        _tc_c_body,
        grid=(_N // _RB,),
        in_specs=[
            pl.BlockSpec((1, _RB, _DH), lambda i: (0, i, 0)),
            pl.BlockSpec((1, _RB, _DH), lambda i: (1, i, 0)),
            pl.BlockSpec((1, _RB, _DH), lambda i: (0, i, 0)),
            pl.BlockSpec((1, _RB, _DH), lambda i: (1, i, 0)),
            pl.BlockSpec((_RB, _NW), lambda i: (i, 0)),
        ],
        out_specs=pl.BlockSpec((_RB, _D), lambda i: (i, 0)),
        out_shape=jax.ShapeDtypeStruct((_N, _D), jnp.float32),
    )(aggp, aggp, ysplit, ysplit, degpt)


def kernel(x, edge_index, W, b):
    ei = edge_index.astype(jnp.int32)
    row = ei[0]
    col = ei[1]
    degp, rowp = _deg_call(row, col)
    degpt = degp.T[:_N]
    ysplit = _tc_a(x, W, b.reshape(1, _D), degpt)
    rowp3 = rowp.reshape(_NS, _NCH, _CHW)
    col3 = col.reshape(_NS, _NCH, _CHW)
    (aggp,) = _agg_call(ysplit, rowp3, col3)
    return _tc_c(aggp, ysplit, degpt)


# speedup vs baseline: 1.3292x; 1.3292x over previous
import jax
import jax.numpy as jnp
from jax.experimental import pallas as pl
from jax.experimental.pallas import tpu as pltpu
from jax.experimental.pallas import tpu_sc as plsc

_N = 10000
_E = 320000
_D = 128

_NC = 2
_NS = 16
_NW = _NC * _NS
_EPT = _E // _NW
_VCH = _EPT // 16
_DEGN = 10240
_CHW = 125
_EPS = _E // _NS
_NCH = _EPS // _CHW
_NBUF = 4
_DH = _D // _NC
_NACC = 10240
_STRIPE = _NACC // _NS
_ZR = 64

_MIN_NORM = 1e-15
_MAXN = 1.0 - 4e-3


def _sc_mesh():
    return plsc.VectorSubcoreMesh(
        core_axis_name="c", subcore_axis_name="s", num_cores=_NC, num_subcores=_NS
    )


def _deg_body(row_hbm, col_hbm, degp_hbm, rowp_hbm, rowbuf, colbuf, degbuf):
    cid = jax.lax.axis_index("c")
    sid = jax.lax.axis_index("s")
    wid = sid * _NC + cid
    base = wid * _EPT
    pltpu.sync_copy(row_hbm.at[pl.ds(base, _EPT)], rowbuf)
    pltpu.sync_copy(col_hbm.at[pl.ds(base, _EPT)], colbuf)

    def zero(i, carry):
        degbuf[pl.ds(i * 16, 16)] = jnp.zeros((16,), jnp.float32)
        return carry

    jax.lax.fori_loop(0, _DEGN // 16, zero, 0)

    ones = jnp.ones((16,), jnp.float32)
    nvec = jnp.full((16,), _N, jnp.int32)

    def step(i, carry):
        r = rowbuf[pl.ds(i * 16, 16)]
        c = colbuf[pl.ds(i * 16, 16)]
        m = r != c
        plsc.addupdate_scatter(degbuf, [r], ones, mask=m)
        rowbuf[pl.ds(i * 16, 16)] = jnp.where(m, r, nvec)
        return carry

    jax.lax.fori_loop(0, _VCH, step, 0)

    pltpu.sync_copy(degbuf, degp_hbm.at[wid])
    pltpu.sync_copy(rowbuf, rowp_hbm.at[pl.ds(base, _EPT)])


def _deg_call(row, col):
    f = pl.kernel(
        _deg_body,
        out_type=[
            jax.ShapeDtypeStruct((_NW, _DEGN), jnp.float32),
            jax.ShapeDtypeStruct((_E,), jnp.int32),
        ],
        mesh=_sc_mesh(),
        scratch_types=[
            pltpu.VMEM((_EPT,), jnp.int32),
            pltpu.VMEM((_EPT,), jnp.int32),
            pltpu.VMEM((_DEGN,), jnp.float32),
        ],
        compiler_params=pltpu.CompilerParams(needs_layout_passes=False),
    )
    return f(row, col)


def _agg_body(y_hbm, rowp_hbm, col_hbm, out_hbm,
              rowbuf, colbuf, bufs, zbuf, acc, gsem, ssem):
    cid = jax.lax.axis_index("c")
    sid = jax.lax.axis_index("s")
    pltpu.sync_copy(rowp_hbm.at[sid], rowbuf)
    pltpu.sync_copy(col_hbm.at[sid], colbuf)

    def zrow(i, carry):
        def zcol(j, carry2):
            zbuf[i, pl.ds(j * 16, 16)] = jnp.zeros((16,), jnp.float32)
            return carry2
        return jax.lax.fori_loop(0, _DH // 16, zcol, carry)

    jax.lax.fori_loop(0, _ZR, zrow, 0)
    for k in range(_STRIPE // _ZR):
        pltpu.sync_copy(zbuf, acc.at[pl.ds(sid * _STRIPE + k * _ZR, _ZR)])
    plsc.subcore_barrier()

    def gather(g, j):
        return pltpu.make_async_copy(
            y_hbm.at[cid].at[colbuf.at[g]], bufs[j], gsem.at[j]
        )

    def scat(g, j):
        pltpu.sync_copy(bufs[j], acc.at[rowbuf.at[g]], add=True)

    gather(0, 0).start()
    gather(1, 1).start()

    def body(h, carry):
        g0 = _NBUF * h
        for j in range(_NBUF):
            g = g0 + j
            jn = (j + 2) % _NBUF

            @pl.when(g + 2 < _NCH)
            def _():
                gather(g + 2, jn).start()

            gather(g, j).wait()
            scat(g, j)

        return carry

    jax.lax.fori_loop(0, _NCH // _NBUF, body, 0)

    plsc.subcore_barrier()
    pltpu.sync_copy(
        acc.at[pl.ds(sid * _STRIPE, _STRIPE)],
        out_hbm.at[cid, pl.ds(sid * _STRIPE, _STRIPE)],
    )


def _agg_call(ysplit, rowp, col):
    f = pl.kernel(
        _agg_body,
        out_type=[jax.ShapeDtypeStruct((_NC, _NACC, _DH), jnp.float32)],
        mesh=_sc_mesh(),
        scratch_types=[
            pltpu.VMEM((_NCH, _CHW), jnp.int32),
            pltpu.VMEM((_NCH, _CHW), jnp.int32),
            [pltpu.VMEM((_CHW, _DH), jnp.float32) for _ in range(_NBUF)],
            pltpu.VMEM((_ZR, _DH), jnp.float32),
            pltpu.VMEM_SHARED((_NACC, _DH), jnp.float32),
            pltpu.SemaphoreType.DMA((_NBUF,)),
            pltpu.SemaphoreType.DMA((_NBUF,)),
        ],
        compiler_params=pltpu.CompilerParams(
            needs_layout_passes=False, use_tc_tiling_on_sc=False
        ),
    )
    return f(ysplit, rowp, col)


def _nrm(v):
    return jnp.sqrt(jnp.clip(jnp.sum(v * v, axis=-1, keepdims=True), 1e-30, None))


def _artanh(v):
    v = jnp.clip(v, -1.0 + 1e-7, 1.0 - 1e-7)
    return 0.5 * (jnp.log1p(v) - jnp.log1p(-v))


def _proj(v):
    n = jnp.clip(_nrm(v), _MIN_NORM, None)
    return jnp.where(n > _MAXN, v / n * _MAXN, v)


def _expmap0(u):
    un = jnp.clip(_nrm(u), _MIN_NORM, None)
    return jnp.tanh(un) * u / un


def _logmap0(p):
    pn = jnp.clip(_nrm(p), _MIN_NORM, None)
    return _artanh(pn) * p / pn


def _mobius_add(a, bb):
    a2 = jnp.sum(a * a, axis=-1, keepdims=True)
    b2 = jnp.sum(bb * bb, axis=-1, keepdims=True)
    ab = jnp.sum(a * bb, axis=-1, keepdims=True)
    num = (1.0 + 2.0 * ab + b2) * a + (1.0 - a2) * bb
    den = 1.0 + 2.0 * ab + a2 * b2
    return num / jnp.clip(den, _MIN_NORM, None)


_RB = 1000


def _tc_a_body(x_ref, w_ref, b_ref, degpt_ref, y_ref):
    x = x_ref[...]
    w = w_ref[...]
    b = b_ref[...]
    mx = jax.lax.dot_general(
        x, w, (((1,), (1,)), ((), ())), preferred_element_type=jnp.float32
    )
    xn = jnp.clip(_nrm(x), _MIN_NORM, None)
    mxn = jnp.clip(_nrm(mx), _MIN_NORM, None)
    res = jnp.tanh(mxn / xn * _artanh(xn)) * mx / mxn
    iszero = jnp.sum(jnp.abs(mx), axis=-1, keepdims=True) == 0.0
    res = jnp.where(iszero, jnp.zeros_like(res), res)
    res = _proj(res)
    hb = _proj(_expmap0(b))
    h = _proj(_mobius_add(res, hb))
    xt = _logmap0(h)
    deg = jnp.sum(degpt_ref[...], axis=1, keepdims=True) + 1.0
    dis = jax.lax.rsqrt(deg)
    y = dis * xt
    y_ref[...] = jnp.stack([y[:, :_DH], y[:, _DH:]], axis=0)


def _tc_a(x, w, b2, degpt):
    return pl.pallas_call(
        _tc_a_body,
        grid=(_N // _RB,),
        in_specs=[
            pl.BlockSpec((_RB, _D), lambda i: (i, 0)),
            pl.BlockSpec((_D, _D), lambda i: (0, 0)),
            pl.BlockSpec((1, _D), lambda i: (0, 0)),
            pl.BlockSpec((_RB, _NW), lambda i: (i, 0)),
        ],
        out_specs=pl.BlockSpec((_NC, _RB, _DH), lambda i: (0, i, 0)),
        out_shape=jax.ShapeDtypeStruct((_NC, _N, _DH), jnp.float32),
    )(x, w, b2, degpt)


def _tc_c_body(a0_ref, a1_ref, y0_ref, y1_ref, degpt_ref, o_ref):
    s = (jnp.concatenate([a0_ref[0], a1_ref[0]], axis=-1)
         + jnp.concatenate([y0_ref[0], y1_ref[0]], axis=-1))
    deg = jnp.sum(degpt_ref[...], axis=1, keepdims=True) + 1.0
    dis = jax.lax.rsqrt(deg)
    st = dis * s
    o1 = _proj(_expmap0(st))
    a = _logmap0(o1)
    a = jnp.where(a >= 0.0, a, 0.01 * a)
    o_ref[...] = _proj(_expmap0(a))


def _tc_c(aggp, ysplit, degpt):
    return pl.pallas_call(
        _tc_c_body,
        grid=(_N // _RB,),
        in_specs=[
            pl.BlockSpec((1, _RB, _DH), lambda i: (0, i, 0)),
            pl.BlockSpec((1, _RB, _DH), lambda i: (1, i, 0)),
            pl.BlockSpec((1, _RB, _DH), lambda i: (0, i, 0)),
            pl.BlockSpec((1, _RB, _DH), lambda i: (1, i, 0)),
            pl.BlockSpec((_RB, _NW), lambda i: (i, 0)),
        ],
        out_specs=pl.BlockSpec((_RB, _D), lambda i: (i, 0)),
        out_shape=jax.ShapeDtypeStruct((_N, _D), jnp.float32),
    )(aggp, aggp, ysplit, ysplit, degpt)


def kernel(x, edge_index, W, b):
    ei = edge_index.astype(jnp.int32)
    row = ei[0]
    col = ei[1]
    degp, rowp = _deg_call(row, col)
    degpt = degp.T[:_N]
    ysplit = _tc_a(x, W, b.reshape(1, _D), degpt)
    rowp3 = rowp.reshape(_NS, _NCH, _CHW)
    col3 = col.reshape(_NS, _NCH, _CHW)
    (aggp,) = _agg_call(ysplit, rowp3, col3)
    return _tc_c(aggp, ysplit, degpt)


# speedup vs baseline: 1.3534x; 1.0182x over previous
import jax
import jax.numpy as jnp
from jax.experimental import pallas as pl
from jax.experimental.pallas import tpu as pltpu
from jax.experimental.pallas import tpu_sc as plsc

_N = 10000
_E = 320000
_D = 128

_NC = 2
_NS = 16
_NW = _NC * _NS
_EPT = _E // _NW
_VCH = _EPT // 16
_DEGN = 10240
_CHW = 125
_EPS = _E // _NS
_NCH = _EPS // _CHW
_NBUF = 4
_DH = _D // _NC
_NACC = 10240
_STRIPE = _NACC // _NS
_ZR = 64

_MIN_NORM = 1e-15
_MAXN = 1.0 - 4e-3


def _sc_mesh():
    return plsc.VectorSubcoreMesh(
        core_axis_name="c", subcore_axis_name="s", num_cores=_NC, num_subcores=_NS
    )


def _deg_body(row_hbm, col_hbm, degp_hbm, rowp_hbm, rowbuf, colbuf, degbuf):
    cid = jax.lax.axis_index("c")
    sid = jax.lax.axis_index("s")
    wid = sid * _NC + cid
    base = wid * _EPT
    pltpu.sync_copy(row_hbm.at[pl.ds(base, _EPT)], rowbuf)
    pltpu.sync_copy(col_hbm.at[pl.ds(base, _EPT)], colbuf)

    def zero(i, carry):
        degbuf[pl.ds(i * 16, 16)] = jnp.zeros((16,), jnp.float32)
        return carry

    jax.lax.fori_loop(0, _DEGN // 16, zero, 0)

    ones = jnp.ones((16,), jnp.float32)
    nvec = jnp.full((16,), _N, jnp.int32)

    def step(i, carry):
        r = rowbuf[pl.ds(i * 16, 16)]
        c = colbuf[pl.ds(i * 16, 16)]
        m = r != c
        plsc.addupdate_scatter(degbuf, [r], ones, mask=m)
        rowbuf[pl.ds(i * 16, 16)] = jnp.where(m, r, nvec)
        return carry

    jax.lax.fori_loop(0, _VCH, step, 0)

    pltpu.sync_copy(degbuf, degp_hbm.at[wid])
    pltpu.sync_copy(rowbuf, rowp_hbm.at[pl.ds(base, _EPT)])


def _deg_call(row, col):
    f = pl.kernel(
        _deg_body,
        out_type=[
            jax.ShapeDtypeStruct((_NW, _DEGN), jnp.float32),
            jax.ShapeDtypeStruct((_E,), jnp.int32),
        ],
        mesh=_sc_mesh(),
        scratch_types=[
            pltpu.VMEM((_EPT,), jnp.int32),
            pltpu.VMEM((_EPT,), jnp.int32),
            pltpu.VMEM((_DEGN,), jnp.float32),
        ],
        compiler_params=pltpu.CompilerParams(needs_layout_passes=False),
    )
    return f(row, col)


def _agg_body(y_hbm, rowp_hbm, col_hbm, out_hbm,
              rowbuf, colbuf, bufs, zbuf, acc, gsem, ssem):
    cid = jax.lax.axis_index("c")
    sid = jax.lax.axis_index("s")
    pltpu.sync_copy(rowp_hbm.at[sid], rowbuf)
    pltpu.sync_copy(col_hbm.at[sid], colbuf)

    def zrow(i, carry):
        def zcol(j, carry2):
            zbuf[i, pl.ds(j * 16, 16)] = jnp.zeros((16,), jnp.float32)
            return carry2
        return jax.lax.fori_loop(0, _DH // 16, zcol, carry)

    jax.lax.fori_loop(0, _ZR, zrow, 0)
    for k in range(_STRIPE // _ZR):
        pltpu.sync_copy(zbuf, acc.at[pl.ds(sid * _STRIPE + k * _ZR, _ZR)])
    plsc.subcore_barrier()

    def gather(g, j):
        return pltpu.make_async_copy(
            y_hbm.at[cid].at[colbuf.at[g]], bufs[j], gsem.at[j]
        )

    def scat(g, j):
        pltpu.sync_copy(bufs[j], acc.at[rowbuf.at[g]], add=True)

    gather(0, 0).start()
    gather(1, 1).start()
    gather(2, 2).start()

    def body(h, carry):
        g0 = _NBUF * h
        for j in range(_NBUF):
            g = g0 + j
            jn = (j + 3) % _NBUF

            @pl.when(g + 3 < _NCH)
            def _():
                gather(g + 3, jn).start()

            gather(g, j).wait()
            scat(g, j)

        return carry

    jax.lax.fori_loop(0, _NCH // _NBUF, body, 0)

    plsc.subcore_barrier()
    pltpu.sync_copy(
        acc.at[pl.ds(sid * _STRIPE, _STRIPE)],
        out_hbm.at[cid, pl.ds(sid * _STRIPE, _STRIPE)],
    )


def _agg_call(ysplit, rowp, col):
    f = pl.kernel(
        _agg_body,
        out_type=[jax.ShapeDtypeStruct((_NC, _NACC, _DH), jnp.float32)],
        mesh=_sc_mesh(),
        scratch_types=[
            pltpu.VMEM((_NCH, _CHW), jnp.int32),
            pltpu.VMEM((_NCH, _CHW), jnp.int32),
            [pltpu.VMEM((_CHW, _DH), jnp.float32) for _ in range(_NBUF)],
            pltpu.VMEM((_ZR, _DH), jnp.float32),
            pltpu.VMEM_SHARED((_NACC, _DH), jnp.float32),
            pltpu.SemaphoreType.DMA((_NBUF,)),
            pltpu.SemaphoreType.DMA((_NBUF,)),
        ],
        compiler_params=pltpu.CompilerParams(
            needs_layout_passes=False, use_tc_tiling_on_sc=False
        ),
    )
    return f(ysplit, rowp, col)


def _nrm(v):
    return jnp.sqrt(jnp.clip(jnp.sum(v * v, axis=-1, keepdims=True), 1e-30, None))


def _artanh(v):
    v = jnp.clip(v, -1.0 + 1e-7, 1.0 - 1e-7)
    return 0.5 * (jnp.log1p(v) - jnp.log1p(-v))


def _proj(v):
    n = jnp.clip(_nrm(v), _MIN_NORM, None)
    return jnp.where(n > _MAXN, v / n * _MAXN, v)


def _expmap0(u):
    un = jnp.clip(_nrm(u), _MIN_NORM, None)
    return jnp.tanh(un) * u / un


def _logmap0(p):
    pn = jnp.clip(_nrm(p), _MIN_NORM, None)
    return _artanh(pn) * p / pn


def _mobius_add(a, bb):
    a2 = jnp.sum(a * a, axis=-1, keepdims=True)
    b2 = jnp.sum(bb * bb, axis=-1, keepdims=True)
    ab = jnp.sum(a * bb, axis=-1, keepdims=True)
    num = (1.0 + 2.0 * ab + b2) * a + (1.0 - a2) * bb
    den = 1.0 + 2.0 * ab + a2 * b2
    return num / jnp.clip(den, _MIN_NORM, None)


_RB = 1000


def _tc_a_body(x_ref, w_ref, b_ref, degpt_ref, y_ref):
    x = x_ref[...]
    w = w_ref[...]
    b = b_ref[...]
    mx = jax.lax.dot_general(
        x, w, (((1,), (1,)), ((), ())), preferred_element_type=jnp.float32
    )
    xn = jnp.clip(_nrm(x), _MIN_NORM, None)
    mxn = jnp.clip(_nrm(mx), _MIN_NORM, None)
    res = jnp.tanh(mxn / xn * _artanh(xn)) * mx / mxn
    iszero = jnp.sum(jnp.abs(mx), axis=-1, keepdims=True) == 0.0
    res = jnp.where(iszero, jnp.zeros_like(res), res)
    res = _proj(res)
    hb = _proj(_expmap0(b))
    h = _proj(_mobius_add(res, hb))
    xt = _logmap0(h)
    deg = jnp.sum(degpt_ref[...], axis=1, keepdims=True) + 1.0
    dis = jax.lax.rsqrt(deg)
    y = dis * xt
    y_ref[...] = jnp.stack([y[:, :_DH], y[:, _DH:]], axis=0)


def _tc_a(x, w, b2, degpt):
    return pl.pallas_call(
        _tc_a_body,
        grid=(_N // _RB,),
        in_specs=[
            pl.BlockSpec((_RB, _D), lambda i: (i, 0)),
            pl.BlockSpec((_D, _D), lambda i: (0, 0)),
            pl.BlockSpec((1, _D), lambda i: (0, 0)),
            pl.BlockSpec((_RB, _NW), lambda i: (i, 0)),
        ],
        out_specs=pl.BlockSpec((_NC, _RB, _DH), lambda i: (0, i, 0)),
        out_shape=jax.ShapeDtypeStruct((_NC, _N, _DH), jnp.float32),
    )(x, w, b2, degpt)


def _tc_c_body(a0_ref, a1_ref, y0_ref, y1_ref, degpt_ref, o_ref):
    s = (jnp.concatenate([a0_ref[0], a1_ref[0]], axis=-1)
         + jnp.concatenate([y0_ref[0], y1_ref[0]], axis=-1))
    deg = jnp.sum(degpt_ref[...], axis=1, keepdims=True) + 1.0
    dis = jax.lax.rsqrt(deg)
    st = dis * s
    o1 = _proj(_expmap0(st))
    a = _logmap0(o1)
    a = jnp.where(a >= 0.0, a, 0.01 * a)
    o_ref[...] = _proj(_expmap0(a))


def _tc_c(aggp, ysplit, degpt):
    return pl.pallas_call(
        _tc_c_body,
        grid=(_N // _RB,),
        in_specs=[
            pl.BlockSpec((1, _RB, _DH), lambda i: (0, i, 0)),
            pl.BlockSpec((1, _RB, _DH), lambda i: (1, i, 0)),
            pl.BlockSpec((1, _RB, _DH), lambda i: (0, i, 0)),
            pl.BlockSpec((1, _RB, _DH), lambda i: (1, i, 0)),
            pl.BlockSpec((_RB, _NW), lambda i: (i, 0)),
        ],
        out_specs=pl.BlockSpec((_RB, _D), lambda i: (i, 0)),
        out_shape=jax.ShapeDtypeStruct((_N, _D), jnp.float32),
    )(aggp, aggp, ysplit, ysplit, degpt)


def kernel(x, edge_index, W, b):
    ei = edge_index.astype(jnp.int32)
    row = ei[0]
    col = ei[1]
    degp, rowp = _deg_call(row, col)
    degpt = degp.T[:_N]
    ysplit = _tc_a(x, W, b.reshape(1, _D), degpt)
    rowp3 = rowp.reshape(_NS, _NCH, _CHW)
    col3 = col.reshape(_NS, _NCH, _CHW)
    (aggp,) = _agg_call(ysplit, rowp3, col3)
    return _tc_c(aggp, ysplit, degpt)
